# baseline (speedup 1.0000x reference)
import functools

import jax
import jax.numpy as jnp
import numpy as np
from jax import lax
from jax.experimental import pallas as pl
from jax.experimental.pallas import tpu as pltpu
from jax.experimental.pallas import tpu_sc as plsc

N = 10000
E = 320000
D = 128
DE = 16
H = 4
HD = D // H
CONCAT = 2 * D + DE
E2 = 2 * E
PW = D + DE

NC = 2
NS = 16
NW = NC * NS

PT_G = E2 // NW
CG = 400
NCH_G = PT_G // CG

PT_S = E // NW
CS = 80
NCH_S = PT_S // CS
NP = 10240
NPS = NP // NS

EB = 1000
NB = 2000


def _gelu(v):
    return 0.5 * v * (1.0 + lax.erf(v * np.float32(0.7071067811865476)))


def _ln_rows(v, w, b, eps=1e-5):
    mu = jnp.mean(v, axis=-1, keepdims=True)
    var = jnp.mean((v - mu) ** 2, axis=-1, keepdims=True)
    return (v - mu) / jnp.sqrt(var + eps) * w + b


def _ln_x_body(x_ref, w_ref, b_ref, o_ref):
    o_ref[...] = _ln_rows(x_ref[...], w_ref[...], b_ref[...]).astype(jnp.bfloat16)


def _ln_x(x, w, b):
    grid = N // NB
    return pl.pallas_call(
        _ln_x_body,
        grid=(grid,),
        in_specs=[
            pl.BlockSpec((NB, D), lambda i: (i, 0)),
            pl.BlockSpec((1, D), lambda i: (0, 0)),
            pl.BlockSpec((1, D), lambda i: (0, 0)),
        ],
        out_specs=pl.BlockSpec((NB, D), lambda i: (i, 0)),
        out_shape=jax.ShapeDtypeStruct((N, D), jnp.bfloat16),
    )(x, w.reshape(1, D), b.reshape(1, D))


def _sc_gather(xn, idx_all):
    mesh = plsc.VectorSubcoreMesh(core_axis_name="c", subcore_axis_name="s")

    @functools.partial(
        pl.kernel,
        out_type=jax.ShapeDtypeStruct((E2, D // 2), jnp.int32),
        mesh=mesh,
        scratch_types=[
            pltpu.VMEM((PT_G,), jnp.int32),
            pltpu.VMEM((CG, D // 2), jnp.int32),
            pltpu.SemaphoreType.DMA,
        ],
        compiler_params=pltpu.CompilerParams(use_tc_tiling_on_sc=False),
    )
    def k(xn_hbm, idx_hbm, out_hbm, idx_v, rows_v, sem):
        wid = lax.axis_index("c") * NS + lax.axis_index("s")
        base = wid * PT_G
        pltpu.sync_copy(idx_hbm.at[pl.ds(base, PT_G)], idx_v)

        def body(i, carry):
            pltpu.async_copy(
                xn_hbm.at[idx_v.at[pl.ds(i * CG, CG)]], rows_v, sem
            ).wait()
            pltpu.sync_copy(rows_v, out_hbm.at[pl.ds(base + i * CG, CG)])
            return carry

        lax.fori_loop(0, NCH_G, body, 0)

    return k(xn, idx_all)


def _edge_body(src_ref, tgt_ref, ea_ref,
               en0w_ref, en0b_ref,
               w1s_ref, w1t_ref, w1e_ref, b1_ref,
               w2n_ref, b2n_ref, w2r_ref, b2r_ref, w2e_ref, b2e_ref,
               r0_ref, r1_ref, p0_ref, p1_ref,
               efw_ref, efb_ref, en1w_ref, en1b_ref,
               pay_s_ref, pay_t_ref, ne_ref):
    f32 = jnp.float32
    bf16 = jnp.bfloat16
    ea = ea_ref[...]
    ean = _ln_rows(ea, en0w_ref[...], en0b_ref[...])
    pre = (
        jnp.dot(src_ref[...], w1s_ref[...], preferred_element_type=f32)
        + jnp.dot(tgt_ref[...], w1t_ref[...], preferred_element_type=f32)
        + jnp.dot(ean.astype(bf16), w1e_ref[...], preferred_element_type=f32)
        + b1_ref[...]
    )
    h = _gelu(pre).astype(bf16)
    on = jnp.dot(h, w2n_ref[...], preferred_element_type=f32) + b2n_ref[...]
    orr = jnp.dot(h, w2r_ref[...], preferred_element_type=f32) + b2r_ref[...]
    oe = jnp.dot(h, w2e_ref[...], preferred_element_type=f32) + b2e_ref[...]
    ex = jnp.exp(orr)
    rep0 = jnp.dot(ex, r0_ref[...], preferred_element_type=f32)
    rep1 = jnp.dot(ex, r1_ref[...], preferred_element_type=f32)
    pay_s_ref[...] = jnp.concatenate(
        [on[:, :D] * rep0, jnp.dot(ex, p0_ref[...], preferred_element_type=f32)],
        axis=1)
    pay_t_ref[...] = jnp.concatenate(
        [on[:, D:] * rep1, jnp.dot(ex, p1_ref[...], preferred_element_type=f32)],
        axis=1)
    lne = _ln_rows(oe, en1w_ref[...], en1b_ref[...])
    ne_ref[...] = _gelu(
        jnp.dot(lne, efw_ref[...], preferred_element_type=f32) + efb_ref[...]
    ) + ea


def _edge_mlp(gathered, edge_attr, en0w, en0b, w1s, w1t, w1e, b1,
              w2n, b2n, w2r, b2r, w2e, b2e, r0, r1, p0, p1,
              efw, efb, en1w, en1b):
    grid = E // EB
    full = lambda r, c: pl.BlockSpec((r, c), lambda i: (0, 0))
    out_shapes = (
        jax.ShapeDtypeStruct((E, PW), jnp.float32),
        jax.ShapeDtypeStruct((E, PW), jnp.float32),
        jax.ShapeDtypeStruct((E, DE), jnp.float32),
    )
    return pl.pallas_call(
        _edge_body,
        grid=(grid,),
        in_specs=[
            pl.BlockSpec((EB, D), lambda i: (i, 0)),
            pl.BlockSpec((EB, D), lambda i: (E // EB + i, 0)),
            pl.BlockSpec((EB, DE), lambda i: (i, 0)),
            full(1, DE), full(1, DE),
            full(D, CONCAT), full(D, CONCAT), full(DE, CONCAT),
            full(1, CONCAT),
            full(CONCAT, 2 * D), full(1, 2 * D),
            full(CONCAT, 2 * H), full(1, 2 * H),
            full(CONCAT, DE), full(1, DE),
            full(2 * H, D), full(2 * H, D),
            full(2 * H, DE), full(2 * H, DE),
            full(DE, DE), full(1, DE),
            full(1, DE), full(1, DE),
        ],
        out_specs=(
            pl.BlockSpec((EB, PW), lambda i: (i, 0)),
            pl.BlockSpec((EB, PW), lambda i: (i, 0)),
            pl.BlockSpec((EB, DE), lambda i: (i, 0)),
        ),
        out_shape=out_shapes,
        compiler_params=pltpu.CompilerParams(
            dimension_semantics=("arbitrary",),
        ),
    )(gathered, gathered, edge_attr, en0w, en0b, w1s, w1t, w1e, b1,
      w2n, b2n, w2r, b2r, w2e, b2e, r0, r1, p0, p1, efw, efb, en1w, en1b)


def _sc_scatter(pay_s, pay_t, src2d, tgt2d, zn):
    mesh = plsc.VectorSubcoreMesh(core_axis_name="c", subcore_axis_name="s")

    @functools.partial(
        pl.kernel,
        out_type=jax.ShapeDtypeStruct((2 * NP, PW), jnp.float32),
        mesh=mesh,
        scratch_types=[
            pltpu.VMEM((NCH_S, CS), jnp.int32),
            pltpu.VMEM((CS, PW), jnp.float32),
            pltpu.VMEM_SHARED((NP, PW), jnp.float32),
        ],
        compiler_params=pltpu.CompilerParams(use_tc_tiling_on_sc=False),
    )
    def k(ps_hbm, pt_hbm, si_hbm, ti_hbm, zn_hbm,
          acc_out, idx_v, pay_v, acc_sh):
        c = lax.axis_index("c")
        s = lax.axis_index("s")
        wid = c * NS + s
        rz = s * NPS
        pltpu.sync_copy(zn_hbm.at[pl.ds(rz, NPS)], acc_sh.at[pl.ds(rz, NPS)])
        pltpu.sync_copy(si_hbm.at[wid], idx_v)
        plsc.subcore_barrier()
        ebase = wid * PT_S

        def body_s(i, carry):
            pltpu.sync_copy(ps_hbm.at[pl.ds(ebase + i * CS, CS)], pay_v)
            pltpu.sync_copy(pay_v, acc_sh.at[idx_v.at[i]], add=True)
            return carry

        lax.fori_loop(0, NCH_S, body_s, 0)
        pltpu.sync_copy(ti_hbm.at[wid], idx_v)

        def body_t(i, carry):
            pltpu.sync_copy(pt_hbm.at[pl.ds(ebase + i * CS, CS)], pay_v)
            pltpu.sync_copy(pay_v, acc_sh.at[idx_v.at[i]], add=True)
            return carry

        lax.fori_loop(0, NCH_S, body_t, 0)
        plsc.subcore_barrier()
        ro = c * NP + rz
        pltpu.sync_copy(acc_sh.at[pl.ds(rz, NPS)], acc_out.at[pl.ds(ro, NPS)])

    return k(pay_s, pay_t, src2d, tgt2d, zn)


def _node_body(p0_ref, p1_ref, x_ref, rp_ref,
               nn1w_ref, nn1b_ref, fw_ref, fb_ref, o_ref):
    p0 = p0_ref[...]
    p1 = p1_ref[...]
    num = p0[:, :D] + p1[:, :D]
    den4 = p0[:, D:D + H] + p1[:, D:D + H] + 1e-16
    den_rep = jnp.dot(den4, rp_ref[...], preferred_element_type=jnp.float32)
    raw = num / den_rep
    lnn = _ln_rows(raw, nn1w_ref[...], nn1b_ref[...])
    o_ref[...] = _gelu(
        jnp.dot(lnn, fw_ref[...], preferred_element_type=jnp.float32)
        + fb_ref[...]
    ) + x_ref[...]


def _node_update(part0, part1, x, rp, nn1w, nn1b, fw, fb):
    grid = N // NB
    full = lambda r, c: pl.BlockSpec((r, c), lambda i: (0, 0))
    return pl.pallas_call(
        _node_body,
        grid=(grid,),
        in_specs=[
            pl.BlockSpec((NB, PW), lambda i: (i, 0)),
            pl.BlockSpec((NB, PW), lambda i: (i, 0)),
            pl.BlockSpec((NB, D), lambda i: (i, 0)),
            full(H, D), full(1, D), full(1, D), full(D, D), full(1, D),
        ],
        out_specs=pl.BlockSpec((NB, D), lambda i: (i, 0)),
        out_shape=jax.ShapeDtypeStruct((N, D), jnp.float32),
    )(part0, part1, x, rp, nn1w, nn1b, fw, fb)


def kernel(x, edge_index, edge_attr,
           attn_fc1_w, attn_fc1_b, attn_fc2_w, attn_fc2_b,
           node_fc1_w, node_fc1_b, edge_fc1_w, edge_fc1_b,
           nn0_w, nn0_b, en0_w, en0_b, nn1_w, nn1_b, en1_w, en1_b):
    f32 = jnp.float32
    src_idx = edge_index[0]
    tgt_idx = edge_index[1]

    w1s = attn_fc1_w[:D]
    w1t = attn_fc1_w[D:2 * D]
    w1e = attn_fc1_w[2 * D:]
    b1 = attn_fc1_b.reshape(1, CONCAT)
    w2r = attn_fc2_w[:, :2 * H]
    w2n = attn_fc2_w[:, 2 * H:2 * H + 2 * D]
    w2e = attn_fc2_w[:, 2 * H + 2 * D:]
    b2r = attn_fc2_b[:2 * H].reshape(1, 2 * H)
    b2n = attn_fc2_b[2 * H:2 * H + 2 * D].reshape(1, 2 * D)
    b2e = attn_fc2_b[2 * H + 2 * D:].reshape(1, DE)

    r0 = np.zeros((2 * H, D), np.float32)
    r1 = np.zeros((2 * H, D), np.float32)
    p0 = np.zeros((2 * H, DE), np.float32)
    p1 = np.zeros((2 * H, DE), np.float32)
    for h in range(H):
        r0[h, h * HD:(h + 1) * HD] = 1.0
        r1[H + h, h * HD:(h + 1) * HD] = 1.0
        p0[h, h] = 1.0
        p1[H + h, h] = 1.0
    rp = np.zeros((H, D), np.float32)
    for h in range(H):
        rp[h, h * HD:(h + 1) * HD] = 1.0
    r0, r1, p0, p1, rp = (jnp.asarray(a) for a in (r0, r1, p0, p1, rp))

    xn = _ln_x(x, nn0_w, nn0_b)

    idx_all = jnp.concatenate([src_idx, tgt_idx])
    xn_i32 = lax.bitcast_convert_type(xn.reshape(N, D // 2, 2), jnp.int32)
    gath_i32 = _sc_gather(xn_i32, idx_all)
    gathered = lax.bitcast_convert_type(gath_i32, jnp.bfloat16).reshape(E2, D)

    bf16 = jnp.bfloat16
    pay_s, pay_t, new_edges = _edge_mlp(
        gathered, edge_attr,
        en0_w.reshape(1, DE), en0_b.reshape(1, DE),
        w1s.astype(bf16), w1t.astype(bf16), w1e.astype(bf16), b1,
        w2n.astype(bf16), b2n, w2r.astype(bf16), b2r, w2e.astype(bf16), b2e,
        r0, r1, p0, p1,
        edge_fc1_w, edge_fc1_b.reshape(1, DE),
        en1_w.reshape(1, DE), en1_b.reshape(1, DE),
    )

    src2d = src_idx.reshape(NW, NCH_S, CS)
    tgt2d = tgt_idx.reshape(NW, NCH_S, CS)
    zn = jnp.zeros((NP, PW), f32)
    acc_part = _sc_scatter(pay_s, pay_t, src2d, tgt2d, zn)

    new_nodes = _node_update(
        acc_part[:N], acc_part[NP:NP + N], x,
        rp, nn1_w.reshape(1, D), nn1_b.reshape(1, D),
        node_fc1_w, node_fc1_b.reshape(1, D))

    return (new_nodes, new_edges)


# speedup vs baseline: 2.0408x; 2.0408x over previous
import functools

import jax
import jax.numpy as jnp
import numpy as np
from jax import lax
from jax.experimental import pallas as pl
from jax.experimental.pallas import tpu as pltpu
from jax.experimental.pallas import tpu_sc as plsc

N = 10000
E = 320000
D = 128
DE = 16
H = 4
HD = D // H
CONCAT = 2 * D + DE
E2 = 2 * E
PW = D + DE

NC = 2
NS = 16
NW = NC * NS

KS = 2
ECS = E // KS

PT_G = 2 * ECS // NW
CG = 400
NCH_G = PT_G // CG

PT_S = ECS // NW
CS = 40
NCH_S = PT_S // CS
NP = 10240
NPS = NP // NS

EB = 2000
NB = 2000


def _gelu(v):
    return 0.5 * v * (1.0 + lax.erf(v * np.float32(0.7071067811865476)))


def _ln_rows(v, w, b, eps=1e-5):
    mu = jnp.mean(v, axis=-1, keepdims=True)
    var = jnp.mean((v - mu) ** 2, axis=-1, keepdims=True)
    return (v - mu) / jnp.sqrt(var + eps) * w + b


def _ln_x_body(x_ref, w_ref, b_ref, o_ref):
    o_ref[...] = _ln_rows(x_ref[...], w_ref[...], b_ref[...])


def _ln_x(x, w, b):
    grid = N // NB
    return pl.pallas_call(
        _ln_x_body,
        grid=(grid,),
        in_specs=[
            pl.BlockSpec((NB, D), lambda i: (i, 0)),
            pl.BlockSpec((1, D), lambda i: (0, 0)),
            pl.BlockSpec((1, D), lambda i: (0, 0)),
        ],
        out_specs=pl.BlockSpec((NB, D), lambda i: (i, 0)),
        out_shape=jax.ShapeDtypeStruct((N, D), jnp.float32),
    )(x, w.reshape(1, D), b.reshape(1, D))


def _sc_gather(xn, idx_all):
    mesh = plsc.VectorSubcoreMesh(core_axis_name="c", subcore_axis_name="s")

    @functools.partial(
        pl.kernel,
        out_type=jax.ShapeDtypeStruct((2 * ECS, D), jnp.float32),
        mesh=mesh,
        scratch_types=[
            pltpu.VMEM((PT_G,), jnp.int32),
            pltpu.VMEM((2, CG, D), jnp.float32),
            pltpu.SemaphoreType.DMA,
            pltpu.SemaphoreType.DMA,
        ],
    )
    def k(xn_hbm, idx_hbm, out_hbm, idx_v, rows_v, sem0, sem1):
        wid = lax.axis_index("c") * NS + lax.axis_index("s")
        base = wid * PT_G
        pltpu.sync_copy(idx_hbm.at[pl.ds(base, PT_G)], idx_v)
        sems = (sem0, sem1)

        def start(i, b):
            pltpu.async_copy(
                xn_hbm.at[idx_v.at[pl.ds(i * CG, CG)]], rows_v.at[b], sems[b])

        def finish(i, b):
            pltpu.make_async_copy(
                xn_hbm.at[idx_v.at[pl.ds(0, CG)]], rows_v.at[b], sems[b]
            ).wait()
            pltpu.sync_copy(rows_v.at[b], out_hbm.at[pl.ds(base + i * CG, CG)])

        start(0, 0)

        def body(k2, carry):
            i0 = 2 * k2
            start(i0 + 1, 1)
            finish(i0, 0)
            start(i0 + 2, 0)
            finish(i0 + 1, 1)
            return carry

        lax.fori_loop(0, NCH_G // 2, body, 0)
        finish(NCH_G - 1, 0)

    return k(xn, idx_all)


def _edge_body(src_ref, tgt_ref, ea_ref,
               en0w_ref, en0b_ref,
               w1s_ref, w1t_ref, w1e_ref, b1_ref,
               w2n_ref, b2n_ref, w2r_ref, b2r_ref, w2e_ref, b2e_ref,
               r0_ref, r1_ref, p0_ref, p1_ref,
               efw_ref, efb_ref, en1w_ref, en1b_ref,
               pay_s_ref, pay_t_ref, ne_ref):
    f32 = jnp.float32
    ea = ea_ref[...]
    ean = _ln_rows(ea, en0w_ref[...], en0b_ref[...])
    pre = (
        jnp.dot(src_ref[...], w1s_ref[...], preferred_element_type=f32)
        + jnp.dot(tgt_ref[...], w1t_ref[...], preferred_element_type=f32)
        + jnp.dot(ean, w1e_ref[...], preferred_element_type=f32)
        + b1_ref[...]
    )
    h = _gelu(pre)
    on = jnp.dot(h, w2n_ref[...], preferred_element_type=f32) + b2n_ref[...]
    orr = jnp.dot(h, w2r_ref[...], preferred_element_type=f32) + b2r_ref[...]
    oe = jnp.dot(h, w2e_ref[...], preferred_element_type=f32) + b2e_ref[...]
    ex = jnp.exp(orr)
    rep0 = jnp.dot(ex, r0_ref[...], preferred_element_type=f32)
    rep1 = jnp.dot(ex, r1_ref[...], preferred_element_type=f32)
    pay_s_ref[...] = jnp.concatenate(
        [on[:, :D] * rep0, jnp.dot(ex, p0_ref[...], preferred_element_type=f32)],
        axis=1)
    pay_t_ref[...] = jnp.concatenate(
        [on[:, D:] * rep1, jnp.dot(ex, p1_ref[...], preferred_element_type=f32)],
        axis=1)
    lne = _ln_rows(oe, en1w_ref[...], en1b_ref[...])
    ne_ref[...] = _gelu(
        jnp.dot(lne, efw_ref[...], preferred_element_type=f32) + efb_ref[...]
    ) + ea


def _edge_mlp(gathered, edge_attr, en0w, en0b, w1s, w1t, w1e, b1,
              w2n, b2n, w2r, b2r, w2e, b2e, r0, r1, p0, p1,
              efw, efb, en1w, en1b):
    grid = ECS // EB
    full = lambda r, c: pl.BlockSpec((r, c), lambda i: (0, 0))
    out_shapes = (
        jax.ShapeDtypeStruct((ECS, PW), jnp.float32),
        jax.ShapeDtypeStruct((ECS, PW), jnp.float32),
        jax.ShapeDtypeStruct((ECS, DE), jnp.float32),
    )
    return pl.pallas_call(
        _edge_body,
        grid=(grid,),
        in_specs=[
            pl.BlockSpec((EB, D), lambda i: (i, 0)),
            pl.BlockSpec((EB, D), lambda i: (ECS // EB + i, 0)),
            pl.BlockSpec((EB, DE), lambda i: (i, 0)),
            full(1, DE), full(1, DE),
            full(D, CONCAT), full(D, CONCAT), full(DE, CONCAT),
            full(1, CONCAT),
            full(CONCAT, 2 * D), full(1, 2 * D),
            full(CONCAT, 2 * H), full(1, 2 * H),
            full(CONCAT, DE), full(1, DE),
            full(2 * H, D), full(2 * H, D),
            full(2 * H, DE), full(2 * H, DE),
            full(DE, DE), full(1, DE),
            full(1, DE), full(1, DE),
        ],
        out_specs=(
            pl.BlockSpec((EB, PW), lambda i: (i, 0)),
            pl.BlockSpec((EB, PW), lambda i: (i, 0)),
            pl.BlockSpec((EB, DE), lambda i: (i, 0)),
        ),
        out_shape=out_shapes,
        compiler_params=pltpu.CompilerParams(
            dimension_semantics=("arbitrary",),
        ),
    )(gathered, gathered, edge_attr, en0w, en0b, w1s, w1t, w1e, b1,
      w2n, b2n, w2r, b2r, w2e, b2e, r0, r1, p0, p1, efw, efb, en1w, en1b)


def _sc_scatter(pay_s, pay_t, src2d, tgt2d, zn):
    mesh = plsc.VectorSubcoreMesh(core_axis_name="c", subcore_axis_name="s")

    @functools.partial(
        pl.kernel,
        out_type=jax.ShapeDtypeStruct((2 * NP, PW), jnp.float32),
        mesh=mesh,
        scratch_types=[
            pltpu.VMEM((NCH_S, CS), jnp.int32),
            pltpu.VMEM((2, CS, PW), jnp.float32),
            pltpu.VMEM_SHARED((NP, PW), jnp.float32),
            pltpu.SemaphoreType.DMA,
            pltpu.SemaphoreType.DMA,
        ],
        compiler_params=pltpu.CompilerParams(use_tc_tiling_on_sc=False),
    )
    def k(ps_hbm, pt_hbm, si_hbm, ti_hbm, zn_hbm,
          acc_out, idx_v, pay_v, acc_sh, sem0, sem1):
        c = lax.axis_index("c")
        s = lax.axis_index("s")
        wid = c * NS + s
        rz = s * NPS
        pltpu.sync_copy(zn_hbm.at[pl.ds(rz, NPS)], acc_sh.at[pl.ds(rz, NPS)])
        pltpu.sync_copy(si_hbm.at[wid], idx_v)
        plsc.subcore_barrier()
        ebase = wid * PT_S
        sems = (sem0, sem1)

        def phase(pay_hbm):
            def start(i, b):
                pltpu.async_copy(
                    pay_hbm.at[pl.ds(ebase + i * CS, CS)], pay_v.at[b], sems[b])

            def finish(i, b):
                pltpu.make_async_copy(
                    pay_hbm.at[pl.ds(ebase, CS)], pay_v.at[b], sems[b]
                ).wait()
                pltpu.sync_copy(pay_v.at[b], acc_sh.at[idx_v.at[i]], add=True)

            start(0, 0)

            def body(k2, carry):
                i0 = 2 * k2
                start(i0 + 1, 1)
                finish(i0, 0)
                start(i0 + 2, 0)
                finish(i0 + 1, 1)
                return carry

            lax.fori_loop(0, NCH_S // 2, body, 0)
            finish(NCH_S - 1, 0)

        phase(ps_hbm)
        pltpu.sync_copy(ti_hbm.at[wid], idx_v)
        phase(pt_hbm)
        plsc.subcore_barrier()
        ro = c * NP + rz
        pltpu.sync_copy(acc_sh.at[pl.ds(rz, NPS)], acc_out.at[pl.ds(ro, NPS)])

    return k(pay_s, pay_t, src2d, tgt2d, zn)


def _node_body(p0_ref, p1_ref, p2_ref, p3_ref, x_ref, rp_ref,
               nn1w_ref, nn1b_ref, fw_ref, fb_ref, o_ref):
    p = p0_ref[...] + p1_ref[...] + p2_ref[...] + p3_ref[...]
    num = p[:, :D]
    den4 = p[:, D:D + H] + 1e-16
    den_rep = jnp.dot(den4, rp_ref[...], preferred_element_type=jnp.float32)
    raw = num / den_rep
    lnn = _ln_rows(raw, nn1w_ref[...], nn1b_ref[...])
    o_ref[...] = _gelu(
        jnp.dot(lnn, fw_ref[...], preferred_element_type=jnp.float32)
        + fb_ref[...]
    ) + x_ref[...]


def _node_update(parts, x, rp, nn1w, nn1b, fw, fb):
    grid = N // NB
    full = lambda r, c: pl.BlockSpec((r, c), lambda i: (0, 0))
    return pl.pallas_call(
        _node_body,
        grid=(grid,),
        in_specs=[
            pl.BlockSpec((NB, PW), lambda i: (i, 0)),
            pl.BlockSpec((NB, PW), lambda i: (i, 0)),
            pl.BlockSpec((NB, PW), lambda i: (i, 0)),
            pl.BlockSpec((NB, PW), lambda i: (i, 0)),
            pl.BlockSpec((NB, D), lambda i: (i, 0)),
            full(H, D), full(1, D), full(1, D), full(D, D), full(1, D),
        ],
        out_specs=pl.BlockSpec((NB, D), lambda i: (i, 0)),
        out_shape=jax.ShapeDtypeStruct((N, D), jnp.float32),
    )(*parts, x, rp, nn1w, nn1b, fw, fb)


def kernel(x, edge_index, edge_attr,
           attn_fc1_w, attn_fc1_b, attn_fc2_w, attn_fc2_b,
           node_fc1_w, node_fc1_b, edge_fc1_w, edge_fc1_b,
           nn0_w, nn0_b, en0_w, en0_b, nn1_w, nn1_b, en1_w, en1_b):
    f32 = jnp.float32
    src_idx = edge_index[0]
    tgt_idx = edge_index[1]

    w1s = attn_fc1_w[:D]
    w1t = attn_fc1_w[D:2 * D]
    w1e = attn_fc1_w[2 * D:]
    b1 = attn_fc1_b.reshape(1, CONCAT)
    w2r = attn_fc2_w[:, :2 * H]
    w2n = attn_fc2_w[:, 2 * H:2 * H + 2 * D]
    w2e = attn_fc2_w[:, 2 * H + 2 * D:]
    b2r = attn_fc2_b[:2 * H].reshape(1, 2 * H)
    b2n = attn_fc2_b[2 * H:2 * H + 2 * D].reshape(1, 2 * D)
    b2e = attn_fc2_b[2 * H + 2 * D:].reshape(1, DE)

    r0 = np.zeros((2 * H, D), np.float32)
    r1 = np.zeros((2 * H, D), np.float32)
    p0 = np.zeros((2 * H, DE), np.float32)
    p1 = np.zeros((2 * H, DE), np.float32)
    for h in range(H):
        r0[h, h * HD:(h + 1) * HD] = 1.0
        r1[H + h, h * HD:(h + 1) * HD] = 1.0
        p0[h, h] = 1.0
        p1[H + h, h] = 1.0
    rp = np.zeros((H, D), np.float32)
    for h in range(H):
        rp[h, h * HD:(h + 1) * HD] = 1.0
    r0, r1, p0, p1, rp = (jnp.asarray(a) for a in (r0, r1, p0, p1, rp))

    xn = _ln_x(x, nn0_w, nn0_b)

    zn = jnp.zeros((NP, PW), f32)
    parts = []
    nes = []
    for k in range(KS):
        sl = slice(k * ECS, (k + 1) * ECS)
        idx_sl = jnp.concatenate([src_idx[sl], tgt_idx[sl]])
        gathered = _sc_gather(xn, idx_sl)
        ps, pt, ne = _edge_mlp(
            gathered, edge_attr[sl],
            en0_w.reshape(1, DE), en0_b.reshape(1, DE),
            w1s, w1t, w1e, b1, w2n, b2n, w2r, b2r, w2e, b2e,
            r0, r1, p0, p1,
            edge_fc1_w, edge_fc1_b.reshape(1, DE),
            en1_w.reshape(1, DE), en1_b.reshape(1, DE),
        )
        acc = _sc_scatter(
            ps, pt,
            src_idx[sl].reshape(NW, NCH_S, CS),
            tgt_idx[sl].reshape(NW, NCH_S, CS), zn)
        parts += [acc[:N], acc[NP:NP + N]]
        nes.append(ne)

    new_edges = jnp.concatenate(nes) if KS > 1 else nes[0]

    new_nodes = _node_update(
        parts, x,
        rp, nn1_w.reshape(1, D), nn1_b.reshape(1, D),
        node_fc1_w, node_fc1_b.reshape(1, D))

    return (new_nodes, new_edges)
